# baseline (speedup 1.0000x reference)
import functools

import jax
import jax.numpy as jnp
from jax import lax
from jax.experimental import pallas as pl
from jax.experimental.pallas import tpu as pltpu
from jax.experimental.pallas import tpu_sc as plsc

N = 50000
E = 800000
DIM = 64
NG = 64
CUTOFF = 5.0
NCONV = 3
NGRAPHS = 1000
NTYPES = 100

NCORES = 2
NTILES = 16
E_PAD = 819200
EPT = E_PAD // NTILES
WIN = 256
NWIN = EPT // WIN
HALF = DIM // 2
TRASH = 48
SH_ROWS = 50048
ZSTRIPE = SH_ROWS // NTILES
OUT_STRIPE = 3128


def _ssp(x):
    return jnp.log1p(jnp.exp(x)) - jnp.log(2.0)


def _sc_conv_body(h0_hbm, h1_hbm, f0_hbm, f1_hbm, sd_hbm, a0_hbm, a1_hbm,
                  sd0, sd1, f_v, rows0, rows1, agg_sh,
                  fsem, gsem0, gsem1, sdsem0, sdsem1):
    c = lax.axis_index("c")
    s = lax.axis_index("s")
    nchunk = WIN // 128

    zero = jnp.zeros((16,), jnp.float32)

    @pl.loop(0, WIN)
    def _zero_rows(r):
        rows0[r, pl.ds(0, 16)] = zero
        rows0[r, pl.ds(16, 16)] = zero

    zbase = s * ZSTRIPE
    nz = ZSTRIPE // WIN

    @pl.loop(0, nz)
    def _zero_stripe(t):
        pltpu.sync_copy(rows0, agg_sh.at[pl.ds(zbase + t * WIN, WIN)])

    pltpu.sync_copy(rows0.at[pl.ds(0, ZSTRIPE - nz * WIN)],
                    agg_sh.at[pl.ds(zbase + nz * WIN, ZSTRIPE - nz * WIN)])
    plsc.subcore_barrier()

    tile_w0 = s * NWIN

    def sd_rows(w):
        wc = jnp.minimum(w, NWIN - 1)
        return pl.ds((tile_w0 + wc) * 2 * nchunk, 2 * nchunk)

    def f_rows(w):
        wc = jnp.minimum(w, NWIN - 1)
        return pl.ds((tile_w0 + wc) * WIN, WIN)

    def issue_f(w):
        @pl.when(c == 0)
        def _f0():
            pltpu.async_copy(f0_hbm.at[f_rows(w)], f_v, fsem)

        @pl.when(c == 1)
        def _f1():
            pltpu.async_copy(f1_hbm.at[f_rows(w)], f_v, fsem)

    def issue_gathers(sd_v, rows_v, gsem):
        @pl.when(c == 0)
        def _g0():
            for j in range(nchunk):
                pltpu.async_copy(h0_hbm.at[sd_v.at[2 * j]],
                                 rows_v.at[pl.ds(j * 128, 128)], gsem)

        @pl.when(c == 1)
        def _g1():
            for j in range(nchunk):
                pltpu.async_copy(h1_hbm.at[sd_v.at[2 * j]],
                                 rows_v.at[pl.ds(j * 128, 128)], gsem)

    def wait_gathers(sd_v, rows_v, gsem):
        for j in range(nchunk):
            pltpu.make_async_copy(h0_hbm.at[sd_v.at[2 * j]],
                                  rows_v.at[pl.ds(j * 128, 128)], gsem).wait()

    pltpu.sync_copy(sd_hbm.at[sd_rows(0)], sd0)
    issue_f(0)
    issue_gathers(sd0, rows0, gsem0)
    pltpu.async_copy(sd_hbm.at[sd_rows(1)], sd1, sdsem1)

    def half(w, sdA, sdB, rowsA, rowsB, gsemA, gsemB, sdsemA, sdsemB):
        pltpu.make_async_copy(f0_hbm.at[f_rows(w)], f_v, fsem).wait()
        wait_gathers(sdA, rowsA, gsemA)

        @pl.loop(0, WIN)
        def _mul(r):
            rowsA[r, pl.ds(0, 16)] = rowsA[r, pl.ds(0, 16)] * f_v[r, pl.ds(0, 16)]
            rowsA[r, pl.ds(16, 16)] = rowsA[r, pl.ds(16, 16)] * f_v[r, pl.ds(16, 16)]

        issue_f(w + 1)
        pltpu.make_async_copy(sd_hbm.at[sd_rows(w + 1)], sdB, sdsemB).wait()
        issue_gathers(sdB, rowsB, gsemB)
        for j in range(nchunk):
            pltpu.sync_copy(rowsA.at[pl.ds(j * 128, 128)],
                            agg_sh.at[sdA.at[2 * j + 1]], add=True)
        pltpu.async_copy(sd_hbm.at[sd_rows(w + 2)], sdA, sdsemA)

    @pl.loop(0, NWIN, step=2)
    def _window(w):
        half(w, sd0, sd1, rows0, rows1, gsem0, gsem1, sdsem0, sdsem1)
        half(w + 1, sd1, sd0, rows1, rows0, gsem1, gsem0, sdsem1, sdsem0)

    pltpu.make_async_copy(f0_hbm.at[f_rows(NWIN)], f_v, fsem).wait()
    wait_gathers(sd0, rows0, gsem0)
    pltpu.make_async_copy(sd_hbm.at[sd_rows(NWIN)], sd1, sdsem1).wait()

    plsc.subcore_barrier()
    obase = s * OUT_STRIPE
    last = N - (NTILES - 1) * OUT_STRIPE

    @pl.when(c == 0)
    def _out0():
        @pl.when(s < NTILES - 1)
        def _full():
            pltpu.sync_copy(agg_sh.at[pl.ds(obase, OUT_STRIPE)],
                            a0_hbm.at[pl.ds(obase, OUT_STRIPE)])

        @pl.when(s == NTILES - 1)
        def _last():
            pltpu.sync_copy(agg_sh.at[pl.ds(obase, last)],
                            a0_hbm.at[pl.ds(obase, last)])

    @pl.when(c == 1)
    def _out1():
        @pl.when(s < NTILES - 1)
        def _full():
            pltpu.sync_copy(agg_sh.at[pl.ds(obase, OUT_STRIPE)],
                            a1_hbm.at[pl.ds(obase, OUT_STRIPE)])

        @pl.when(s == NTILES - 1)
        def _last():
            pltpu.sync_copy(agg_sh.at[pl.ds(obase, last)],
                            a1_hbm.at[pl.ds(obase, last)])


@jax.jit
def _sc_conv(h0, h1, f0, f1, sd):
    mesh = plsc.VectorSubcoreMesh(core_axis_name="c", subcore_axis_name="s")
    f = pl.kernel(
        _sc_conv_body,
        out_type=(jax.ShapeDtypeStruct((N, HALF), jnp.float32),
                  jax.ShapeDtypeStruct((N, HALF), jnp.float32)),
        mesh=mesh,
        compiler_params=pltpu.CompilerParams(use_tc_tiling_on_sc=False),
        scratch_types=[
            pltpu.VMEM((2 * (WIN // 128), 128), jnp.int32),
            pltpu.VMEM((2 * (WIN // 128), 128), jnp.int32),
            pltpu.VMEM((WIN, HALF), jnp.float32),
            pltpu.VMEM((WIN, HALF), jnp.float32),
            pltpu.VMEM((WIN, HALF), jnp.float32),
            pltpu.VMEM_SHARED((SH_ROWS, HALF), jnp.float32),
            pltpu.SemaphoreType.DMA,
            pltpu.SemaphoreType.DMA,
            pltpu.SemaphoreType.DMA,
            pltpu.SemaphoreType.DMA,
            pltpu.SemaphoreType.DMA,
        ],
    )
    return f(h0, h1, f0, f1, sd)


_EB = 4096
_NB = 2000


def _f_body(d_ref, w1_ref, b1_ref, w2_ref, b2_ref, f0_ref, f1_ref):
    d = d_ref[0]
    ones = jnp.ones((1, NG), jnp.float32)
    d2 = lax.dot_general(d, ones, (((0,), (0,)), ((), ())))
    gap = CUTOFF / (NG - 1)
    centers = lax.broadcasted_iota(jnp.int32, (_EB, NG), 1).astype(jnp.float32) * gap
    rbf = jnp.exp(-((d2 - centers) ** 2) / (gap ** 2))
    f1 = _ssp(jnp.dot(rbf, w1_ref[...],
                      preferred_element_type=jnp.float32) + b1_ref[...])
    f2 = _ssp(jnp.dot(f1, w2_ref[...],
                      preferred_element_type=jnp.float32) + b2_ref[...])
    f0_ref[...] = f2[:, :HALF]
    f1_ref[...] = f2[:, HALF:]


def _f_tc(dist3, w1, b1, w2, b2):
    grid = E_PAD // _EB
    return pl.pallas_call(
        _f_body,
        grid=(grid,),
        in_specs=[
            pl.BlockSpec((1, 1, _EB), lambda i: (i, 0, 0)),
            pl.BlockSpec((NG, DIM), lambda i: (0, 0)),
            pl.BlockSpec((DIM,), lambda i: (0,)),
            pl.BlockSpec((DIM, DIM), lambda i: (0, 0)),
            pl.BlockSpec((DIM,), lambda i: (0,)),
        ],
        out_specs=[
            pl.BlockSpec((_EB, HALF), lambda i: (i, 0)),
            pl.BlockSpec((_EB, HALF), lambda i: (i, 0)),
        ],
        out_shape=[
            jax.ShapeDtypeStruct((E_PAD, HALF), jnp.float32),
            jax.ShapeDtypeStruct((E_PAD, HALF), jnp.float32),
        ],
    )(dist3, w1, b1, w2, b2)


def _emb_body(z_ref, emb_ref, wn_ref, bn_ref, x_ref, h0_ref, h1_ref):
    z = z_ref[0]
    tids = lax.broadcasted_iota(jnp.int32, (NTYPES, _NB), 0)
    onehot = (tids == z).astype(jnp.float32)
    x = lax.dot_general(onehot, emb_ref[...], (((0,), (0,)), ((), ())))
    h = jnp.dot(x, wn_ref[...], preferred_element_type=jnp.float32) + bn_ref[...]
    x_ref[...] = x
    h0_ref[...] = h[:, :HALF]
    h1_ref[...] = h[:, HALF:]


def _emb_tc(z3, emb_table, wn, bn):
    grid = N // _NB
    return pl.pallas_call(
        _emb_body,
        grid=(grid,),
        in_specs=[
            pl.BlockSpec((1, 1, _NB), lambda i: (i, 0, 0)),
            pl.BlockSpec((NTYPES, DIM), lambda i: (0, 0)),
            pl.BlockSpec((DIM, DIM), lambda i: (0, 0)),
            pl.BlockSpec((DIM,), lambda i: (0,)),
        ],
        out_specs=[
            pl.BlockSpec((_NB, DIM), lambda i: (i, 0)),
            pl.BlockSpec((_NB, HALF), lambda i: (i, 0)),
            pl.BlockSpec((_NB, HALF), lambda i: (i, 0)),
        ],
        out_shape=[
            jax.ShapeDtypeStruct((N, DIM), jnp.float32),
            jax.ShapeDtypeStruct((N, HALF), jnp.float32),
            jax.ShapeDtypeStruct((N, HALF), jnp.float32),
        ],
    )(z3, emb_table, wn, bn)


def _upd_body(a0_ref, a1_ref, x_ref, wo1_ref, bo1_ref, wo2_ref, bo2_ref,
              wn_ref, bn_ref, xn_ref, h0_ref, h1_ref):
    t = (jnp.dot(a0_ref[...], wo1_ref[:HALF], preferred_element_type=jnp.float32)
         + jnp.dot(a1_ref[...], wo1_ref[HALF:], preferred_element_type=jnp.float32)
         + bo1_ref[...])
    o = jnp.dot(_ssp(t), wo2_ref[...], preferred_element_type=jnp.float32) + bo2_ref[...]
    xn = x_ref[...] + o
    hn = jnp.dot(xn, wn_ref[...], preferred_element_type=jnp.float32) + bn_ref[...]
    xn_ref[...] = xn
    h0_ref[...] = hn[:, :HALF]
    h1_ref[...] = hn[:, HALF:]


def _upd_tc(a0, a1, x, wo1, bo1, wo2, bo2, wn, bn):
    grid = N // _NB
    return pl.pallas_call(
        _upd_body,
        grid=(grid,),
        in_specs=[
            pl.BlockSpec((_NB, HALF), lambda i: (i, 0)),
            pl.BlockSpec((_NB, HALF), lambda i: (i, 0)),
            pl.BlockSpec((_NB, DIM), lambda i: (i, 0)),
            pl.BlockSpec((DIM, DIM), lambda i: (0, 0)),
            pl.BlockSpec((DIM,), lambda i: (0,)),
            pl.BlockSpec((DIM, DIM), lambda i: (0, 0)),
            pl.BlockSpec((DIM,), lambda i: (0,)),
            pl.BlockSpec((DIM, DIM), lambda i: (0, 0)),
            pl.BlockSpec((DIM,), lambda i: (0,)),
        ],
        out_specs=[
            pl.BlockSpec((_NB, DIM), lambda i: (i, 0)),
            pl.BlockSpec((_NB, HALF), lambda i: (i, 0)),
            pl.BlockSpec((_NB, HALF), lambda i: (i, 0)),
        ],
        out_shape=[
            jax.ShapeDtypeStruct((N, DIM), jnp.float32),
            jax.ShapeDtypeStruct((N, HALF), jnp.float32),
            jax.ShapeDtypeStruct((N, HALF), jnp.float32),
        ],
    )(a0, a1, x, wo1, bo1, wo2, bo2, wn, bn)


def kernel(node_z, edge_index, distance, graph_ids, emb_table,
           conv_Wn, conv_bn, conv_Wf1, conv_bf1, conv_Wf2, conv_bf2,
           conv_Wo1, conv_bo1, conv_Wo2, conv_bo2,
           W_a1, b_a1, W_a2, b_a2):
    src = edge_index[0].astype(jnp.int32)
    dst = edge_index[1].astype(jnp.int32)

    pad = E_PAD - E
    padidx = jnp.arange(pad, dtype=jnp.int32)
    src_p = jnp.concatenate([src, padidx % N]).reshape(E_PAD // 128, 128)
    dst_p = jnp.concatenate([dst, N + (padidx % TRASH)]).reshape(E_PAD // 128, 128)
    sd = jnp.stack([src_p, dst_p], axis=1).reshape(2 * E_PAD // 128, 128)

    dist3 = jnp.concatenate([distance.astype(jnp.float32),
                             jnp.zeros((pad,), jnp.float32)]
                            ).reshape(E_PAD // _EB, 1, _EB)
    z3 = node_z.astype(jnp.int32).reshape(N // _NB, 1, _NB)

    x, h0, h1 = _emb_tc(z3, emb_table, conv_Wn[0], conv_bn[0])
    fs = [_f_tc(dist3, conv_Wf1[i], conv_bf1[i], conv_Wf2[i], conv_bf2[i])
          for i in range(NCONV)]
    for i in range(NCONV):
        a0, a1 = _sc_conv(h0, h1, fs[i][0], fs[i][1], sd)
        nxt = (i + 1) % NCONV
        x, h0, h1 = _upd_tc(a0, a1, x, conv_Wo1[i], conv_bo1[i],
                            conv_Wo2[i], conv_bo2[i], conv_Wn[nxt], conv_bn[nxt])

    atom = _ssp(x @ W_a1 + b_a1)
    res = atom @ W_a2 + b_a2
    g_sum = jax.ops.segment_sum(res, graph_ids, num_segments=NGRAPHS)
    counts = jax.ops.segment_sum(jnp.ones((N, 1), dtype=res.dtype),
                                 graph_ids, num_segments=NGRAPHS)
    return g_sum / jnp.maximum(counts, 1.0)
